# baseline (speedup 1.0000x reference)
import functools

import jax
import jax.numpy as jnp
from jax import lax
from jax.experimental import pallas as pl
from jax.experimental.pallas import tpu as pltpu
from jax.experimental.pallas import tpu_sc as plsc

BATCH, SEQ, D = 4, 8192, 1024
NUM_CORES, NUM_SUBCORES = 2, 16
NUM_WORKERS = NUM_CORES * NUM_SUBCORES
SEQ_PER_W = SEQ // NUM_WORKERS
CHUNK = 16
N_CHUNKS = SEQ_PER_W // CHUNK
LANES = 16
UNROLL = 8


def _pe_add_body(x_hbm, pos_hbm, out_hbm,
                 pos0, pos1, xb0, xb1, xb2, xb3,
                 sp0, sp1, si0, si1, si2, si3, so0, so1, so2, so3):
    pos_bufs = (pos0, pos1)
    pos_sems = (sp0, sp1)
    x_bufs = (xb0, xb1, xb2, xb3)
    in_sems = (si0, si1, si2, si3)
    out_sems = (so0, so1, so2, so3)

    wid = lax.axis_index("s") * NUM_CORES + lax.axis_index("c")
    seq0 = wid * SEQ_PER_W

    def pos_cp(c, parity):
        row0 = seq0 + c * CHUNK
        return pltpu.make_async_copy(
            pos_hbm.at[pl.ds(row0, CHUNK), :], pos_bufs[parity],
            pos_sems[parity])

    def in_cp(c, b):
        row0 = seq0 + c * CHUNK
        return pltpu.make_async_copy(
            x_hbm.at[b, pl.ds(row0, CHUNK), :], x_bufs[b], in_sems[b])

    def out_cp(c, b):
        row0 = seq0 + c * CHUNK
        return pltpu.make_async_copy(
            x_bufs[b], out_hbm.at[b, pl.ds(row0, CHUNK), :], out_sems[b])

    def add_chunk(xb, posbuf):
        @plsc.parallel_loop(0, D, step=LANES * UNROLL)
        def _(base):
            for u in range(UNROLL):
                off = base + u * LANES
                for r in range(CHUNK):
                    plsc.addupdate(xb.at[r, pl.ds(off, LANES)],
                                   posbuf[r, pl.ds(off, LANES)])

    def chunk_step(c, parity):
        @pl.when(c + 1 < N_CHUNKS)
        def _():
            pos_cp(c + 1, parity ^ 1).start()

        pos_cp(c, parity).wait()
        for b in range(BATCH):
            in_cp(c, b).wait()
            add_chunk(x_bufs[b], pos_bufs[parity])
            out_cp(c, b).start()
            if b >= 1:
                out_cp(c, b - 1).wait()

                @pl.when(c + 1 < N_CHUNKS)
                def _():
                    in_cp(c + 1, b - 1).start()

        out_cp(c, BATCH - 1).wait()

        @pl.when(c + 1 < N_CHUNKS)
        def _():
            in_cp(c + 1, BATCH - 1).start()

    pos_cp(0, 0).start()
    for b in range(BATCH):
        in_cp(0, b).start()

    def loop_body(c2, _):
        chunk_step(c2 * 2, 0)
        chunk_step(c2 * 2 + 1, 1)
        return 0

    lax.fori_loop(0, N_CHUNKS // 2, loop_body, 0)


@functools.partial(
    pl.kernel,
    mesh=plsc.VectorSubcoreMesh(core_axis_name="c", subcore_axis_name="s"),
    out_type=jax.ShapeDtypeStruct((BATCH, SEQ, D), jnp.float32),
    scratch_types=(
        [pltpu.VMEM((CHUNK, D), jnp.float32)] * 2
        + [pltpu.VMEM((CHUNK, D), jnp.float32)] * 4
        + [pltpu.SemaphoreType.DMA] * 10
    ),
)
def _pe_add(*refs):
    _pe_add_body(*refs)


def kernel(x, pos_table):
    return _pe_add(x, pos_table)


# speedup vs baseline: 1.5585x; 1.5585x over previous
import functools

import jax
import jax.numpy as jnp
from jax import lax
from jax.experimental import pallas as pl
from jax.experimental.pallas import tpu as pltpu
from jax.experimental.pallas import tpu_sc as plsc

BATCH, SEQ, D = 4, 8192, 1024
NUM_CORES, NUM_SUBCORES = 2, 16
NUM_WORKERS = NUM_CORES * NUM_SUBCORES
SEQ_PER_W = SEQ // NUM_WORKERS
CHUNK = 16
N_CHUNKS = SEQ_PER_W // CHUNK
LANES = 16
CHUNK_ELEMS = CHUNK * D
UNROLL = 8


def _pe_add_body(x_hbm, pos_hbm, out_hbm,
                 pos0, pos1, xb0, xb1, xb2, xb3,
                 sp0, sp1, si0, si1, si2, si3, so0, so1, so2, so3):
    pos_bufs = (pos0, pos1)
    pos_sems = (sp0, sp1)
    x_bufs = (xb0, xb1, xb2, xb3)
    in_sems = (si0, si1, si2, si3)
    out_sems = (so0, so1, so2, so3)

    wid = lax.axis_index("s") * NUM_CORES + lax.axis_index("c")
    seq0 = wid * SEQ_PER_W

    def pos_cp(c, parity):
        row0 = seq0 + c * CHUNK
        return pltpu.make_async_copy(
            pos_hbm.at[pl.ds(row0, CHUNK), :], pos_bufs[parity],
            pos_sems[parity])

    def in_cp(c, b):
        row0 = seq0 + c * CHUNK
        return pltpu.make_async_copy(
            x_hbm.at[b, pl.ds(row0, CHUNK), :], x_bufs[b], in_sems[b])

    def out_cp(c, b):
        row0 = seq0 + c * CHUNK
        return pltpu.make_async_copy(
            x_bufs[b], out_hbm.at[b, pl.ds(row0, CHUNK), :], out_sems[b])

    def add_chunk(xb, posbuf):
        @plsc.parallel_loop(0, D, step=LANES)
        def _(off):
            for r in range(CHUNK):
                plsc.addupdate(xb.at[r, pl.ds(off, LANES)],
                               posbuf[r, pl.ds(off, LANES)])

    def chunk_step(c, parity):
        @pl.when(c + 1 < N_CHUNKS)
        def _():
            pos_cp(c + 1, parity ^ 1).start()

        pos_cp(c, parity).wait()
        for b in range(BATCH):
            in_cp(c, b).wait()
            add_chunk(x_bufs[b], pos_bufs[parity])
            out_cp(c, b).start()
            if b >= 1:
                out_cp(c, b - 1).wait()

                @pl.when(c + 1 < N_CHUNKS)
                def _():
                    in_cp(c + 1, b - 1).start()

        out_cp(c, BATCH - 1).wait()

        @pl.when(c + 1 < N_CHUNKS)
        def _():
            in_cp(c + 1, BATCH - 1).start()

    pos_cp(0, 0).start()
    for b in range(BATCH):
        in_cp(0, b).start()

    def loop_body(c2, _):
        chunk_step(c2 * 2, 0)
        chunk_step(c2 * 2 + 1, 1)
        return 0

    lax.fori_loop(0, N_CHUNKS // 2, loop_body, 0)


@functools.partial(
    pl.kernel,
    mesh=plsc.VectorSubcoreMesh(core_axis_name="c", subcore_axis_name="s"),
    out_type=jax.ShapeDtypeStruct((BATCH, SEQ, D), jnp.float32),
    scratch_types=(
        [pltpu.VMEM((CHUNK, D), jnp.float32)] * 2
        + [pltpu.VMEM((CHUNK, D), jnp.float32)] * 4
        + [pltpu.SemaphoreType.DMA] * 10
    ),
)
def _pe_add(*refs):
    _pe_add_body(*refs)


def kernel(x, pos_table):
    return _pe_add(x, pos_table)


# speedup vs baseline: 1.5854x; 1.0172x over previous
import functools

import jax
import jax.numpy as jnp
from jax import lax
from jax.experimental import pallas as pl
from jax.experimental.pallas import tpu as pltpu
from jax.experimental.pallas import tpu_sc as plsc

BATCH, SEQ, D = 4, 8192, 1024
NUM_CORES, NUM_SUBCORES = 2, 16
NUM_WORKERS = NUM_CORES * NUM_SUBCORES
SEQ_PER_W = SEQ // NUM_WORKERS
CHUNK = 8
N_CHUNKS = SEQ_PER_W // CHUNK
LANES = 16


def _pe_add_body(x_hbm, pos_hbm, out_hbm, *rest):
    pos_bufs = rest[0:2]
    xb = (rest[2:4], rest[4:6], rest[6:8], rest[8:10])
    pos_sems = rest[10:12]
    in_sems = (rest[12:14], rest[14:16], rest[16:18], rest[18:20])
    out_sems = (rest[20:22], rest[22:24], rest[24:26], rest[26:28])

    wid = lax.axis_index("s") * NUM_CORES + lax.axis_index("c")
    seq0 = wid * SEQ_PER_W

    def pos_cp(c, q):
        row0 = seq0 + c * CHUNK
        return pltpu.make_async_copy(
            pos_hbm.at[pl.ds(row0, CHUNK), :], pos_bufs[q], pos_sems[q])

    def in_cp(c, b, q):
        row0 = seq0 + c * CHUNK
        return pltpu.make_async_copy(
            x_hbm.at[b, pl.ds(row0, CHUNK), :], xb[b][q], in_sems[b][q])

    def out_cp(c, b, q):
        row0 = seq0 + c * CHUNK
        return pltpu.make_async_copy(
            xb[b][q], out_hbm.at[b, pl.ds(row0, CHUNK), :], out_sems[b][q])

    def add_chunk(buf, posbuf):
        @plsc.parallel_loop(0, D, step=LANES)
        def _(off):
            for r in range(CHUNK):
                plsc.addupdate(buf.at[r, pl.ds(off, LANES)],
                               posbuf[r, pl.ds(off, LANES)])

    def chunk_step(c, q):
        @pl.when(c + 1 < N_CHUNKS)
        def _():
            pos_cp(c + 1, q ^ 1).start()

        pos_cp(c, q).wait()
        for b in range(BATCH):
            in_cp(c, b, q).wait()
            add_chunk(xb[b][q], pos_bufs[q])
            out_cp(c, b, q).start()

            @pl.when(c >= 1)
            def _():
                out_cp(c - 1, b, q ^ 1).wait()

            @pl.when(c + 1 < N_CHUNKS)
            def _():
                in_cp(c + 1, b, q ^ 1).start()

    pos_cp(0, 0).start()
    for b in range(BATCH):
        in_cp(0, b, 0).start()

    def loop_body(c2, _):
        chunk_step(c2 * 2, 0)
        chunk_step(c2 * 2 + 1, 1)
        return 0

    lax.fori_loop(0, N_CHUNKS // 2, loop_body, 0)

    q_last = (N_CHUNKS - 1) % 2
    for b in range(BATCH):
        out_cp(N_CHUNKS - 1, b, q_last).wait()


@functools.partial(
    pl.kernel,
    mesh=plsc.VectorSubcoreMesh(core_axis_name="c", subcore_axis_name="s"),
    out_type=jax.ShapeDtypeStruct((BATCH, SEQ, D), jnp.float32),
    scratch_types=(
        [pltpu.VMEM((CHUNK, D), jnp.float32)] * 2
        + [pltpu.VMEM((CHUNK, D), jnp.float32)] * 8
        + [pltpu.SemaphoreType.DMA] * 2
        + [pltpu.SemaphoreType.DMA] * 8
        + [pltpu.SemaphoreType.DMA] * 8
    ),
)
def _pe_add(*refs):
    _pe_add_body(*refs)


def kernel(x, pos_table):
    return _pe_add(x, pos_table)
